# baseline (speedup 1.0000x reference)
import functools

import jax
import jax.numpy as jnp
from jax import lax
from jax.experimental import pallas as pl
from jax.experimental.pallas import tpu as pltpu
from jax.experimental.pallas import tpu_sc as plsc

N = 10000
E = 320000
D_IN = 128
HID = 128
HEADS = 8

NP = 10240
BN = 1280
NB = NP // BN
NC = 2
NS = 16
L = 16
BLK = 128
NBLKS = E // BLK
ROWS_PER_TILE = NP // NS


def _tc0_body(x_ref, w0_ref, asrc_ref, adst_ref, h0_ref, asT_ref, adT_ref):
    h = pl.program_id(0)
    hb = jnp.dot(x_ref[...], w0_ref[...], preferred_element_type=jnp.float32)
    h0_ref[0] = hb
    sel = lax.broadcasted_iota(jnp.int32, (HEADS, 1), 0) == h
    arow_s = jnp.sum(jnp.where(sel, asrc_ref[...], 0.0), axis=0, keepdims=True)
    arow_d = jnp.sum(jnp.where(sel, adst_ref[...], 0.0), axis=0, keepdims=True)
    asT_ref[0, 0] = jnp.sum(hb * arow_s, axis=1)
    adT_ref[0, 0] = jnp.sum(hb * arow_d, axis=1)


def _tc0(xp, W0, att_src0, att_dst0):
    return pl.pallas_call(
        _tc0_body,
        grid=(HEADS, NB),
        in_specs=[
            pl.BlockSpec((BN, D_IN), lambda h, nb: (nb, 0)),
            pl.BlockSpec((D_IN, HID), lambda h, nb: (0, h)),
            pl.BlockSpec((HEADS, HID), lambda h, nb: (0, 0)),
            pl.BlockSpec((HEADS, HID), lambda h, nb: (0, 0)),
        ],
        out_specs=[
            pl.BlockSpec((1, BN, HID), lambda h, nb: (h, nb, 0)),
            pl.BlockSpec((1, 1, BN), lambda h, nb: (h, 0, nb)),
            pl.BlockSpec((1, 1, BN), lambda h, nb: (h, 0, nb)),
        ],
        out_shape=[
            jax.ShapeDtypeStruct((HEADS, NP, HID), jnp.float32),
            jax.ShapeDtypeStruct((HEADS, 1, NP), jnp.float32),
            jax.ShapeDtypeStruct((HEADS, 1, NP), jnp.float32),
        ],
    )(xp, W0, att_src0, att_dst0)


EPAD = 7680
EP = E + EPAD
NBLKP = EP // BLK
NSLOT = 2
SBE = NSLOT * BLK
WPB = 512
HSUB = 4


def _zero_rows(rows):
    zv = jnp.zeros((L,), jnp.float32)

    def body(r, c):
        for j in range(HID // L):
            rows[r, pl.ds(j * L, L)] = zv
        return c

    lax.fori_loop(0, BLK, body, 0)


def _zero_tab(tab):
    zv = jnp.zeros((L,), jnp.float32)

    def body(i, c):
        tab[pl.ds(i * L, L)] = zv
        return c

    lax.fori_loop(0, NP // L, body, 0)


def _scw_body(nh, src_hbm, dst_hbm, asT_hbm, adT_hbm, wT_hbm,
              asrcF, adstF, srcsb, dstsb, wout):
    cid = lax.axis_index("c")
    sid = lax.axis_index("s")
    wid = sid * NC + cid
    e_per_tile = EP // (NS * NC)
    nsteps = e_per_tile // WPB
    base = wid * e_per_tile

    for sub in range((nh + HSUB - 1) // HSUB):
        heads = list(range(sub * HSUB, min(nh, (sub + 1) * HSUB)))
        for hl, hh in enumerate(heads):
            pltpu.sync_copy(asT_hbm.at[hh], asrcF.at[pl.ds(hl * NP, NP)])
            pltpu.sync_copy(adT_hbm.at[hh], adstF.at[pl.ds(hl * NP, NP)])

        def step(i, c):
            goff = base + i * WPB
            pltpu.sync_copy(src_hbm.at[pl.ds(goff, WPB)], srcsb)
            pltpu.sync_copy(dst_hbm.at[pl.ds(goff, WPB)], dstsb)
            for k in range(WPB // L):
                sl = pl.ds(k * L, L)
                s16 = srcsb[sl]
                d16 = dstsb[sl]
                for hl, _ in enumerate(heads):
                    a1 = plsc.load_gather(asrcF, [s16 + hl * NP])
                    a2 = plsc.load_gather(adstF, [d16 + hl * NP])
                    al = a1 + a2
                    al = jnp.where(al >= 0.0, al, 0.2 * al)
                    wout[hl, sl] = jnp.exp(al)
            for hl, hh in enumerate(heads):
                pltpu.sync_copy(wout.at[hl], wT_hbm.at[hh, pl.ds(goff, WPB)])
            return c

        lax.fori_loop(0, nsteps, step, 0)


def _sc_w(nh, src, dst, asT, adT):
    mesh = plsc.VectorSubcoreMesh(core_axis_name="c", subcore_axis_name="s",
                                  num_cores=NC, num_subcores=NS)
    f = pl.kernel(
        functools.partial(_scw_body, nh),
        out_type=jax.ShapeDtypeStruct((nh, EP), jnp.float32),
        mesh=mesh,
        compiler_params=pltpu.CompilerParams(needs_layout_passes=False),
        scratch_types=[
            pltpu.VMEM((HSUB * NP,), jnp.float32),
            pltpu.VMEM((HSUB * NP,), jnp.float32),
            pltpu.VMEM((WPB,), jnp.int32),
            pltpu.VMEM((WPB,), jnp.int32),
            pltpu.VMEM((HSUB, WPB), jnp.float32),
        ],
    )
    return f(src, dst, asT, adT)


def _head_pass(src_hbm, dst_hbm, feat_hbm, wT_hbm, accum, den_tab,
               srcsb, dstsb, wsb, rows, gidx, dstb, gsem, ssem,
               h, base, nq, row_off):
    def load_sb(blk0):
        goff = blk0 * BLK
        pltpu.sync_copy(src_hbm.at[pl.ds(goff, SBE)], srcsb)
        pltpu.sync_copy(dst_hbm.at[pl.ds(goff, SBE)], dstsb)
        pltpu.sync_copy(wT_hbm.at[h, pl.ds(goff, SBE)], wsb)

    def prep(s):
        off = s * BLK
        for k in range(BLK // L):
            sb_sl = pl.ds(off + k * L, L)
            sl = pl.ds(k * L, L)
            s16 = srcsb[sb_sl]
            d16 = dstsb[sb_sl]
            w16 = wsb[sb_sl]
            gidx[s][sl] = s16 + row_off
            dstb[s][sl] = d16
            plsc.addupdate_scatter(den_tab, [d16], w16)

    def scale(s):
        off = s * BLK

        def sbody(k, c):
            w16 = wsb[pl.ds(off + k * L, L)]
            for ll in range(L):
                r = k * L + ll
                wl = w16[ll]
                for j in range(HID // L):
                    sl2 = pl.ds(j * L, L)
                    rows[s][r, sl2] = rows[s][r, sl2] * wl
            return c

        lax.fori_loop(0, BLK // L, sbody, 0)

    def issue_gather(s):
        pltpu.async_copy(feat_hbm.at[gidx[s]], rows[s], gsem[s])

    def wait_gather(s):
        pltpu.make_async_copy(feat_hbm.at[gidx[s]], rows[s], gsem[s]).wait()

    def issue_scatter(s):
        pltpu.async_copy(rows[s], accum.at[dstb[s]], ssem[s], add=True)

    def wait_scatter(s):
        pltpu.make_async_copy(rows[s], accum.at[dstb[s]], ssem[s]).wait()

    load_sb(base)
    for s in range(NSLOT):
        prep(s)
        issue_gather(s)

    def rnd(j, c):
        for s in range(NSLOT):
            wait_gather(s)
            scale(s)
            issue_scatter(s)

        @pl.when(j + 1 < nq)
        def _():
            load_sb(base + NSLOT * (j + 1))
            for s in range(NSLOT):
                wait_scatter(s)
                prep(s)
                issue_gather(s)

        return c

    lax.fori_loop(0, nq, rnd, 0)
    for s in range(NSLOT):
        wait_scatter(s)


def _sch_body(layer0, src_hbm, dst_hbm, feat_hbm, wT_hbm,
              num_hbm, den_hbm, accum, den_tab, srcsb, dstsb, wsb, *rest):
    rows = rest[0:NSLOT]
    gidx = rest[NSLOT:2 * NSLOT]
    dstb = rest[2 * NSLOT:3 * NSLOT]
    gsem = rest[3 * NSLOT:4 * NSLOT]
    ssem = rest[4 * NSLOT:5 * NSLOT]

    cid = lax.axis_index("c")
    sid = lax.axis_index("s")
    rsl = pl.ds(sid * ROWS_PER_TILE, ROWS_PER_TILE)

    if layer0:
        passes = HEADS // NC
        blocks_per_tile = NBLKP // NS
        base = sid * blocks_per_tile
    else:
        passes = 1
        blocks_per_tile = NBLKP // (NS * NC)
        base = (sid * NC + cid) * blocks_per_tile
    nq = blocks_per_tile // NSLOT

    for hp in range(passes):
        if layer0:
            h = (NC * hp + cid).astype(jnp.int32)
            row_off = h * NP
        else:
            h = jnp.int32(0)
            row_off = jnp.int32(0)
        _zero_rows(rows[0])
        for q in range(ROWS_PER_TILE // BLK):
            pltpu.sync_copy(
                rows[0], accum.at[pl.ds(sid * ROWS_PER_TILE + q * BLK, BLK)])
        _zero_tab(den_tab)
        plsc.subcore_barrier()

        _head_pass(src_hbm, dst_hbm, feat_hbm, wT_hbm, accum, den_tab,
                   srcsb, dstsb, wsb, rows, gidx, dstb, gsem, ssem,
                   h, base, nq, row_off)
        plsc.subcore_barrier()
        if layer0:
            pltpu.sync_copy(accum.at[rsl], num_hbm.at[h, rsl])
            pltpu.sync_copy(den_tab, den_hbm.at[h, sid])
        else:
            pltpu.sync_copy(accum.at[rsl], num_hbm.at[cid, rsl])
            pltpu.sync_copy(den_tab, den_hbm.at[cid, sid])
        plsc.subcore_barrier()


def _sc_head(layer0, src, dst, feat, wT):
    mesh = plsc.VectorSubcoreMesh(core_axis_name="c", subcore_axis_name="s",
                                  num_cores=NC, num_subcores=NS)
    dim0 = HEADS if layer0 else NC
    f = pl.kernel(
        functools.partial(_sch_body, layer0),
        out_type=[
            jax.ShapeDtypeStruct((dim0, NP, HID), jnp.float32),
            jax.ShapeDtypeStruct((dim0, NS, NP), jnp.float32),
        ],
        mesh=mesh,
        compiler_params=pltpu.CompilerParams(needs_layout_passes=False),
        scratch_types=(
            [pltpu.VMEM_SHARED((NP, HID), jnp.float32),
             pltpu.VMEM((NP,), jnp.float32),
             pltpu.VMEM((SBE,), jnp.int32),
             pltpu.VMEM((SBE,), jnp.int32),
             pltpu.VMEM((SBE,), jnp.float32)]
            + [pltpu.VMEM((BLK, HID), jnp.float32)] * NSLOT
            + [pltpu.VMEM((BLK,), jnp.int32)] * NSLOT
            + [pltpu.VMEM((BLK,), jnp.int32)] * NSLOT
            + [pltpu.SemaphoreType.DMA] * (2 * NSLOT)
        ),
    )
    return f(src, dst, feat, wT)


def _tcmid_body(num_ref, den_ref, b0_ref, w1_ref, a1s_ref, a1d_ref,
                h1_ref, asT_ref, adT_ref):
    den = jnp.sum(den_ref[...], axis=1)
    acc = jnp.zeros((BN, HID), jnp.float32)
    for h in range(HEADS):
        v = num_ref[h] / (den[h][:, None] + 1e-16) + b0_ref[h][None, :]
        v = jnp.where(v > 0.0, v, jnp.exp(v) - 1.0)
        acc = acc + jnp.dot(v, w1_ref[h], preferred_element_type=jnp.float32)
    h1_ref[...] = acc
    asT_ref[0] = jnp.sum(acc * a1s_ref[...], axis=1)
    adT_ref[0] = jnp.sum(acc * a1d_ref[...], axis=1)


def _tcmid(num0, den0, b0r, W1r, att_src1, att_dst1):
    return pl.pallas_call(
        _tcmid_body,
        grid=(NB,),
        in_specs=[
            pl.BlockSpec((HEADS, BN, HID), lambda nb: (0, nb, 0)),
            pl.BlockSpec((HEADS, NS, BN), lambda nb: (0, 0, nb)),
            pl.BlockSpec((HEADS, HID), lambda nb: (0, 0)),
            pl.BlockSpec((HEADS, HID, HID), lambda nb: (0, 0, 0)),
            pl.BlockSpec((1, HID), lambda nb: (0, 0)),
            pl.BlockSpec((1, HID), lambda nb: (0, 0)),
        ],
        out_specs=[
            pl.BlockSpec((BN, HID), lambda nb: (nb, 0)),
            pl.BlockSpec((1, BN), lambda nb: (0, nb)),
            pl.BlockSpec((1, BN), lambda nb: (0, nb)),
        ],
        out_shape=[
            jax.ShapeDtypeStruct((NP, HID), jnp.float32),
            jax.ShapeDtypeStruct((1, NP), jnp.float32),
            jax.ShapeDtypeStruct((1, NP), jnp.float32),
        ],
    )(num0, den0, b0r, W1r, att_src1, att_dst1)


def _tcfin_body(num_ref, den_ref, b1_ref, out_ref):
    den = jnp.sum(den_ref[...], axis=(0, 1))
    out_ref[...] = ((num_ref[0] + num_ref[1]) / (den[:, None] + 1e-16)
                    + b1_ref[...])


def _tcfin(num1, den1, b1r):
    return pl.pallas_call(
        _tcfin_body,
        grid=(NB,),
        in_specs=[
            pl.BlockSpec((NC, BN, HID), lambda nb: (0, nb, 0)),
            pl.BlockSpec((NC, NS, BN), lambda nb: (0, 0, nb)),
            pl.BlockSpec((1, HID), lambda nb: (0, 0)),
        ],
        out_specs=pl.BlockSpec((BN, HID), lambda nb: (nb, 0)),
        out_shape=jax.ShapeDtypeStruct((NP, HID), jnp.float32),
    )(num1, den1, b1r)


@jax.jit
def kernel(x, edge_index, W0, att_src0, att_dst0, b0, W1, att_src1, att_dst1, b1):
    xp = jnp.pad(x, ((0, NP - N), (0, 0)))
    src = jnp.pad(edge_index[0], (0, EPAD), constant_values=N)
    dst = jnp.pad(edge_index[1], (0, EPAD), constant_values=N)

    h0T, asT0, adT0 = _tc0(xp, W0, att_src0, att_dst0)
    h0flat = h0T.reshape(HEADS * NP, HID)
    wT0 = _sc_w(HEADS, src, dst,
                asT0.reshape(HEADS, NP), adT0.reshape(HEADS, NP))
    num0, den0 = _sc_head(True, src, dst, h0flat, wT0)

    b0r = b0.reshape(HEADS, HID)
    W1r = W1.reshape(HEADS, HID, HID)
    h1, asT1, adT1 = _tcmid(num0, den0, b0r, W1r, att_src1, att_dst1)

    wT1 = _sc_w(1, src, dst, asT1, adT1)
    num1, den1 = _sc_head(False, src, dst, h1, wT1)
    outp = _tcfin(num1, den1, b1.reshape(1, HID))
    return outp[:N]


# speedup vs baseline: 1.0445x; 1.0445x over previous
import functools

import jax
import jax.numpy as jnp
from jax import lax
from jax.experimental import pallas as pl
from jax.experimental.pallas import tpu as pltpu
from jax.experimental.pallas import tpu_sc as plsc

N = 10000
E = 320000
D_IN = 128
HID = 128
HEADS = 8

NP = 10240
BN = 1280
NB = NP // BN
NC = 2
NS = 16
L = 16
BLK = 128
NBLKS = E // BLK
ROWS_PER_TILE = NP // NS


def _tc0_body(x_ref, w0_ref, asrc_ref, adst_ref, h0_ref, asT_ref, adT_ref):
    h = pl.program_id(0)
    hb = jnp.dot(x_ref[...], w0_ref[...], preferred_element_type=jnp.float32)
    h0_ref[0] = hb
    sel = lax.broadcasted_iota(jnp.int32, (HEADS, 1), 0) == h
    arow_s = jnp.sum(jnp.where(sel, asrc_ref[...], 0.0), axis=0, keepdims=True)
    arow_d = jnp.sum(jnp.where(sel, adst_ref[...], 0.0), axis=0, keepdims=True)
    asT_ref[0, 0] = jnp.sum(hb * arow_s, axis=1)
    adT_ref[0, 0] = jnp.sum(hb * arow_d, axis=1)


def _tc0(xp, W0, att_src0, att_dst0):
    return pl.pallas_call(
        _tc0_body,
        grid=(HEADS, NB),
        in_specs=[
            pl.BlockSpec((BN, D_IN), lambda h, nb: (nb, 0)),
            pl.BlockSpec((D_IN, HID), lambda h, nb: (0, h)),
            pl.BlockSpec((HEADS, HID), lambda h, nb: (0, 0)),
            pl.BlockSpec((HEADS, HID), lambda h, nb: (0, 0)),
        ],
        out_specs=[
            pl.BlockSpec((1, BN, HID), lambda h, nb: (h, nb, 0)),
            pl.BlockSpec((1, 1, BN), lambda h, nb: (h, 0, nb)),
            pl.BlockSpec((1, 1, BN), lambda h, nb: (h, 0, nb)),
        ],
        out_shape=[
            jax.ShapeDtypeStruct((HEADS, NP, HID), jnp.float32),
            jax.ShapeDtypeStruct((HEADS, 1, NP), jnp.float32),
            jax.ShapeDtypeStruct((HEADS, 1, NP), jnp.float32),
        ],
    )(xp, W0, att_src0, att_dst0)


EPAD = 7680
EP = E + EPAD
NBLKP = EP // BLK
NSLOT = 5
HBLK = 64
SBE = NSLOT * HBLK
WPB = 512
HSUB = 4


def _zero_rows(rows):
    zv = jnp.zeros((L,), jnp.float32)

    def body(r, c):
        for j in range(HID // L):
            rows[r, pl.ds(j * L, L)] = zv
        return c

    lax.fori_loop(0, HBLK, body, 0)


def _zero_tab(tab):
    zv = jnp.zeros((L,), jnp.float32)

    def body(i, c):
        tab[pl.ds(i * L, L)] = zv
        return c

    lax.fori_loop(0, NP // L, body, 0)


def _scw_body(nh, src_hbm, dst_hbm, asT_hbm, adT_hbm, wT_hbm, den_hbm,
              asrcF, adstF, denF, srcsb, dstsb, wout):
    cid = lax.axis_index("c")
    sid = lax.axis_index("s")
    wid = sid * NC + cid
    e_per_tile = EP // (NS * NC)
    nsteps = e_per_tile // WPB
    base = wid * e_per_tile

    for sub in range((nh + HSUB - 1) // HSUB):
        heads = list(range(sub * HSUB, min(nh, (sub + 1) * HSUB)))
        for hl, hh in enumerate(heads):
            pltpu.sync_copy(asT_hbm.at[hh], asrcF.at[pl.ds(hl * NP, NP)])
            pltpu.sync_copy(adT_hbm.at[hh], adstF.at[pl.ds(hl * NP, NP)])
        zv = jnp.zeros((L,), jnp.float32)

        def zbody(i, c):
            denF[pl.ds(i * L, L)] = zv
            return c

        lax.fori_loop(0, HSUB * NP // L, zbody, 0)

        def step(i, c):
            goff = base + i * WPB
            pltpu.sync_copy(src_hbm.at[pl.ds(goff, WPB)], srcsb)
            pltpu.sync_copy(dst_hbm.at[pl.ds(goff, WPB)], dstsb)
            for k in range(WPB // L):
                sl = pl.ds(k * L, L)
                s16 = srcsb[sl]
                d16 = dstsb[sl]
                for hl, _ in enumerate(heads):
                    a1 = plsc.load_gather(asrcF, [s16 + hl * NP])
                    a2 = plsc.load_gather(adstF, [d16 + hl * NP])
                    al = a1 + a2
                    al = jnp.where(al >= 0.0, al, 0.2 * al)
                    w16 = jnp.exp(al)
                    wout[hl, sl] = w16
                    plsc.addupdate_scatter(denF, [d16 + hl * NP], w16)
            for hl, hh in enumerate(heads):
                pltpu.sync_copy(wout.at[hl], wT_hbm.at[hh, pl.ds(goff, WPB)])
            return c

        lax.fori_loop(0, nsteps, step, 0)
        for hl, hh in enumerate(heads):
            pltpu.sync_copy(denF.at[pl.ds(hl * NP, NP)],
                            den_hbm.at[hh, wid, 0])


def _sc_w(nh, src, dst, asT, adT):
    mesh = plsc.VectorSubcoreMesh(core_axis_name="c", subcore_axis_name="s",
                                  num_cores=NC, num_subcores=NS)
    f = pl.kernel(
        functools.partial(_scw_body, nh),
        out_type=[jax.ShapeDtypeStruct((nh, EP), jnp.float32),
                  jax.ShapeDtypeStruct((nh, NS * NC, 1, NP), jnp.float32)],
        mesh=mesh,
        compiler_params=pltpu.CompilerParams(needs_layout_passes=False),
        scratch_types=[
            pltpu.VMEM((HSUB * NP,), jnp.float32),
            pltpu.VMEM((HSUB * NP,), jnp.float32),
            pltpu.VMEM((HSUB * NP,), jnp.float32),
            pltpu.VMEM((WPB,), jnp.int32),
            pltpu.VMEM((WPB,), jnp.int32),
            pltpu.VMEM((HSUB, WPB), jnp.float32),
        ],
    )
    return f(src, dst, asT, adT)


def _head_pass(src_hbm, dst_hbm, feat_hbm, wT_hbm, accum,
               srcsb, dstsb, wsb, rows, gidx, dstb, gsem, ssem,
               h, base, nq, row_off):
    def load_sb(blk0):
        goff = blk0 * HBLK
        pltpu.sync_copy(src_hbm.at[pl.ds(goff, SBE)], srcsb)
        pltpu.sync_copy(dst_hbm.at[pl.ds(goff, SBE)], dstsb)
        pltpu.sync_copy(wT_hbm.at[pl.ds(h * EP + goff, SBE)], wsb)

    def prep(s):
        off = s * HBLK
        for k in range(HBLK // L):
            sb_sl = pl.ds(off + k * L, L)
            sl = pl.ds(k * L, L)
            gidx[s][sl] = srcsb[sb_sl] + row_off
            dstb[s][sl] = dstsb[sb_sl]

    def scale(s):
        off = s * HBLK

        def sbody(k, c):
            w16 = wsb[pl.ds(off + k * L, L)]
            for ll in range(L):
                r = k * L + ll
                wl = w16[ll]
                for j in range(HID // L):
                    sl2 = pl.ds(j * L, L)
                    rows[s][r, sl2] = rows[s][r, sl2] * wl
            return c

        lax.fori_loop(0, HBLK // L, sbody, 0)

    def issue_gather(s):
        pltpu.async_copy(feat_hbm.at[gidx[s]], rows[s], gsem[s])

    def wait_gather(s):
        pltpu.make_async_copy(feat_hbm.at[gidx[s]], rows[s], gsem[s]).wait()

    def issue_scatter(s):
        pltpu.async_copy(rows[s], accum.at[dstb[s]], ssem[s], add=True)

    def wait_scatter(s):
        pltpu.make_async_copy(rows[s], accum.at[dstb[s]], ssem[s]).wait()

    load_sb(base)
    for s in range(NSLOT):
        prep(s)
        issue_gather(s)

    def rnd(j, c):
        for s in range(NSLOT):
            wait_gather(s)
            scale(s)
            issue_scatter(s)

        @pl.when(j + 1 < nq)
        def _():
            load_sb(base + NSLOT * (j + 1))
            for s in range(NSLOT):
                wait_scatter(s)
                prep(s)
                issue_gather(s)

        return c

    lax.fori_loop(0, nq, rnd, 0)
    for s in range(NSLOT):
        wait_scatter(s)


def _sch_body(layer0, src_hbm, dst_hbm, feat_hbm, wT_hbm,
              num_hbm, accum, srcsb, dstsb, wsb, *rest):
    rows = rest[0:NSLOT]
    gidx = rest[NSLOT:2 * NSLOT]
    dstb = rest[2 * NSLOT:3 * NSLOT]
    gsem = rest[3 * NSLOT:4 * NSLOT]
    ssem = rest[4 * NSLOT:5 * NSLOT]

    cid = lax.axis_index("c")
    sid = lax.axis_index("s")
    rsl = pl.ds(sid * ROWS_PER_TILE, ROWS_PER_TILE)

    nblk64 = EP // HBLK
    if layer0:
        passes = HEADS // NC
        blocks_per_tile = nblk64 // NS
        base = sid * blocks_per_tile
    else:
        passes = 1
        blocks_per_tile = nblk64 // (NS * NC)
        base = (sid * NC + cid) * blocks_per_tile
    nq = blocks_per_tile // NSLOT

    for hp in range(passes):
        if layer0:
            h = (NC * hp + cid).astype(jnp.int32)
            row_off = h * NP
        else:
            h = jnp.int32(0)
            row_off = jnp.int32(0)
        _zero_rows(rows[0])
        for q in range(ROWS_PER_TILE // HBLK):
            pltpu.sync_copy(
                rows[0], accum.at[pl.ds(sid * ROWS_PER_TILE + q * HBLK, HBLK)])
        plsc.subcore_barrier()

        _head_pass(src_hbm, dst_hbm, feat_hbm, wT_hbm, accum,
                   srcsb, dstsb, wsb, rows, gidx, dstb, gsem, ssem,
                   h, base, nq, row_off)
        plsc.subcore_barrier()
        if layer0:
            pltpu.sync_copy(accum.at[rsl], num_hbm.at[h, rsl])
        else:
            pltpu.sync_copy(accum.at[rsl], num_hbm.at[cid, rsl])
        plsc.subcore_barrier()


def _sc_head(layer0, src, dst, feat, wT):
    mesh = plsc.VectorSubcoreMesh(core_axis_name="c", subcore_axis_name="s",
                                  num_cores=NC, num_subcores=NS)
    dim0 = HEADS if layer0 else NC
    f = pl.kernel(
        functools.partial(_sch_body, layer0),
        out_type=jax.ShapeDtypeStruct((dim0, NP, HID), jnp.float32),
        mesh=mesh,
        compiler_params=pltpu.CompilerParams(needs_layout_passes=False),
        scratch_types=(
            [pltpu.VMEM_SHARED((NP, HID), jnp.float32),
             pltpu.VMEM((SBE,), jnp.int32),
             pltpu.VMEM((SBE,), jnp.int32),
             pltpu.VMEM((SBE,), jnp.float32)]
            + [pltpu.VMEM((HBLK, HID), jnp.float32)] * NSLOT
            + [pltpu.VMEM((HBLK,), jnp.int32)] * NSLOT
            + [pltpu.VMEM((HBLK,), jnp.int32)] * NSLOT
            + [pltpu.SemaphoreType.DMA] * (2 * NSLOT)
        ),
    )
    return f(src, dst, feat, wT.reshape(-1))


def _tcmid_body(num_ref, den_ref, b0_ref, w1_ref, a1s_ref, a1d_ref,
                h1_ref, asT_ref, adT_ref):
    den = jnp.sum(den_ref[...], axis=1)
    acc = jnp.zeros((BN, HID), jnp.float32)
    for h in range(HEADS):
        v = num_ref[h] / (den[h][:, None] + 1e-16) + b0_ref[h][None, :]
        v = jnp.where(v > 0.0, v, jnp.exp(v) - 1.0)
        acc = acc + jnp.dot(v, w1_ref[h], preferred_element_type=jnp.float32)
    h1_ref[...] = acc
    asT_ref[0] = jnp.sum(acc * a1s_ref[...], axis=1)
    adT_ref[0] = jnp.sum(acc * a1d_ref[...], axis=1)


def _tcmid(num0, den0, b0r, W1r, att_src1, att_dst1):
    return pl.pallas_call(
        _tcmid_body,
        grid=(NB,),
        in_specs=[
            pl.BlockSpec((HEADS, BN, HID), lambda nb: (0, nb, 0)),
            pl.BlockSpec((HEADS, NS * NC, BN), lambda nb: (0, 0, nb)),
            pl.BlockSpec((HEADS, HID), lambda nb: (0, 0)),
            pl.BlockSpec((HEADS, HID, HID), lambda nb: (0, 0, 0)),
            pl.BlockSpec((1, HID), lambda nb: (0, 0)),
            pl.BlockSpec((1, HID), lambda nb: (0, 0)),
        ],
        out_specs=[
            pl.BlockSpec((BN, HID), lambda nb: (nb, 0)),
            pl.BlockSpec((1, BN), lambda nb: (0, nb)),
            pl.BlockSpec((1, BN), lambda nb: (0, nb)),
        ],
        out_shape=[
            jax.ShapeDtypeStruct((NP, HID), jnp.float32),
            jax.ShapeDtypeStruct((1, NP), jnp.float32),
            jax.ShapeDtypeStruct((1, NP), jnp.float32),
        ],
    )(num0, den0, b0r, W1r, att_src1, att_dst1)


def _tcfin_body(num_ref, den_ref, b1_ref, out_ref):
    den = jnp.sum(den_ref[...], axis=(0, 1))
    out_ref[...] = ((num_ref[0] + num_ref[1]) / (den[:, None] + 1e-16)
                    + b1_ref[...])


def _tcfin(num1, den1, b1r):
    return pl.pallas_call(
        _tcfin_body,
        grid=(NB,),
        in_specs=[
            pl.BlockSpec((NC, BN, HID), lambda nb: (0, nb, 0)),
            pl.BlockSpec((1, NS * NC, BN), lambda nb: (0, 0, nb)),
            pl.BlockSpec((1, HID), lambda nb: (0, 0)),
        ],
        out_specs=pl.BlockSpec((BN, HID), lambda nb: (nb, 0)),
        out_shape=jax.ShapeDtypeStruct((NP, HID), jnp.float32),
    )(num1, den1, b1r)


@jax.jit
def kernel(x, edge_index, W0, att_src0, att_dst0, b0, W1, att_src1, att_dst1, b1):
    xp = jnp.pad(x, ((0, NP - N), (0, 0)))
    src = jnp.pad(edge_index[0], (0, EPAD), constant_values=N)
    dst = jnp.pad(edge_index[1], (0, EPAD), constant_values=N)

    h0T, asT0, adT0 = _tc0(xp, W0, att_src0, att_dst0)
    h0flat = h0T.reshape(HEADS * NP, HID)
    wT0, den0 = _sc_w(HEADS, src, dst,
                      asT0.reshape(HEADS, NP), adT0.reshape(HEADS, NP))
    num0 = _sc_head(True, src, dst, h0flat, wT0)

    b0r = b0.reshape(HEADS, HID)
    W1r = W1.reshape(HEADS, HID, HID)
    h1, asT1, adT1 = _tcmid(num0, den0.reshape(HEADS, NS * NC, NP),
                            b0r, W1r, att_src1, att_dst1)

    wT1, den1 = _sc_w(1, src, dst, asT1, adT1)
    num1 = _sc_head(False, src, dst, h1, wT1)
    outp = _tcfin(num1, den1.reshape(1, NS * NC, NP), b1.reshape(1, HID))
    return outp[:N]
